# Initial kernel scaffold; baseline (speedup 1.0000x reference)
import jax
import jax.numpy as jnp
from jax.experimental import pallas as pl


def kernel(input, adj, W, a):
    raise NotImplementedError("write your pallas kernel here")


# speedup vs baseline: 1.8947x; 1.8947x over previous
import jax
import jax.numpy as jnp
from jax.experimental import pallas as pl
from jax.experimental.pallas import tpu as pltpu

_ALPHA = 0.2


def _proj_kernel(x_ref, w_ref, a_ref, h_ref, fc_ref, fn_ref):
    h = jnp.dot(x_ref[...], w_ref[...], preferred_element_type=jnp.float32)
    h_ref[...] = h
    f = a_ref.shape[0] // 2
    fn_ref[...] = jnp.dot(h, a_ref[:f, :], preferred_element_type=jnp.float32)
    fc_ref[...] = jnp.dot(h, a_ref[f:, :], preferred_element_type=jnp.float32)


def _attn_kernel(adj_ref, fc_ref, fn_ref, h_ref, out_ref):
    e = fc_ref[...] + fn_ref[...]
    e = jnp.maximum(e, _ALPHA * e)
    maskf = adj_ref[...].astype(jnp.float32)
    m = jnp.max(e, axis=1, keepdims=True)
    p = jnp.exp(e - m) * maskf
    s = jnp.sum(p, axis=1, keepdims=True)
    z = jnp.dot(p, h_ref[...], preferred_element_type=jnp.float32)
    z = z / jnp.where(s > 0, s, 1.0)
    out_ref[...] = jnp.where(z > 0, z, jnp.exp(jnp.minimum(z, 0.0)) - 1.0)


def _gat(x, adj, W, a, bm):
    n, f_in = x.shape
    f_out = W.shape[1]
    n_row = n // bm

    h, fc, fn = pl.pallas_call(
        _proj_kernel,
        grid=(n_row,),
        in_specs=[
            pl.BlockSpec((bm, f_in), lambda i: (i, 0)),
            pl.BlockSpec((f_in, f_out), lambda i: (0, 0)),
            pl.BlockSpec((2 * f_out, 1), lambda i: (0, 0)),
        ],
        out_specs=[
            pl.BlockSpec((bm, f_out), lambda i: (i, 0)),
            pl.BlockSpec((bm, 1), lambda i: (i, 0)),
            pl.BlockSpec((bm, 1), lambda i: (i, 0)),
        ],
        out_shape=[
            jax.ShapeDtypeStruct((n, f_out), jnp.float32),
            jax.ShapeDtypeStruct((n, 1), jnp.float32),
            jax.ShapeDtypeStruct((n, 1), jnp.float32),
        ],
    )(x, W, a)

    fn_row = fn.reshape(1, n)

    out = pl.pallas_call(
        _attn_kernel,
        grid=(n_row,),
        in_specs=[
            pl.BlockSpec((bm, n), lambda i: (i, 0)),
            pl.BlockSpec((bm, 1), lambda i: (i, 0)),
            pl.BlockSpec((1, n), lambda i: (0, 0)),
            pl.BlockSpec((n, f_out), lambda i: (0, 0)),
        ],
        out_specs=pl.BlockSpec((bm, f_out), lambda i: (i, 0)),
        out_shape=jax.ShapeDtypeStruct((n, f_out), jnp.float32),
        compiler_params=pltpu.CompilerParams(
            dimension_semantics=("arbitrary",)),
    )(adj, fc, fn_row, h)
    return out


def _pick_block(n, cap):
    best = 8
    for b in range(8, cap + 1, 8):
        if n % b == 0:
            best = b
    return best


def kernel(input, adj, W, a):
    n = input.shape[0]
    bm = _pick_block(n, 200)
    return _gat(input, adj, W, a, bm)


# speedup vs baseline: 2.3224x; 1.2257x over previous
import jax
import jax.numpy as jnp
from jax.experimental import pallas as pl
from jax.experimental.pallas import tpu as pltpu

_ALPHA = 0.2
_NEG_BIG = -1e30


def _proj_kernel(x_ref, w_ref, a_ref, h_ref, fc_ref, fn_ref, mx_ref):
    i = pl.program_id(0)
    h = jnp.dot(x_ref[...], w_ref[...], preferred_element_type=jnp.float32)
    h_ref[...] = h
    f = a_ref.shape[0] // 2
    fn = jnp.dot(h, a_ref[:f, :], preferred_element_type=jnp.float32)
    fn_ref[...] = fn
    fc_ref[...] = jnp.dot(h, a_ref[f:, :], preferred_element_type=jnp.float32)

    @pl.when(i == 0)
    def _init():
        mx_ref[...] = jnp.full_like(mx_ref, _NEG_BIG)

    mx_ref[...] = jnp.maximum(mx_ref[...], jnp.max(fn))


def _attn_kernel(adj_ref, fc_ref, fn_ref, mx_ref, h_ref, out_ref):
    t = fc_ref[...] + mx_ref[...]
    t = jnp.maximum(t, _ALPHA * t)
    e = fc_ref[...] + fn_ref[...]
    p = (jnp.exp(jnp.maximum(e, _ALPHA * e) - t)
         * adj_ref[...].astype(jnp.float32))
    s = jnp.sum(p, axis=1, keepdims=True)
    z = jnp.dot(p, h_ref[...], preferred_element_type=jnp.float32)
    z = z / jnp.where(s > 0, s, 1.0)
    out_ref[...] = jnp.where(z > 0, z, jnp.exp(jnp.minimum(z, 0.0)) - 1.0)


def _gat(x, adj, W, a, bm):
    n, f_in = x.shape
    f_out = W.shape[1]
    n_row = n // bm

    h, fc, fn, mx = pl.pallas_call(
        _proj_kernel,
        grid=(n_row,),
        in_specs=[
            pl.BlockSpec((bm, f_in), lambda i: (i, 0)),
            pl.BlockSpec((f_in, f_out), lambda i: (0, 0)),
            pl.BlockSpec((2 * f_out, 1), lambda i: (0, 0)),
        ],
        out_specs=[
            pl.BlockSpec((bm, f_out), lambda i: (i, 0)),
            pl.BlockSpec((bm, 1), lambda i: (i, 0)),
            pl.BlockSpec((bm, 1), lambda i: (i, 0)),
            pl.BlockSpec((1, 1), lambda i: (0, 0)),
        ],
        out_shape=[
            jax.ShapeDtypeStruct((n, f_out), jnp.float32),
            jax.ShapeDtypeStruct((n, 1), jnp.float32),
            jax.ShapeDtypeStruct((n, 1), jnp.float32),
            jax.ShapeDtypeStruct((1, 1), jnp.float32),
        ],
        compiler_params=pltpu.CompilerParams(
            dimension_semantics=("arbitrary",)),
    )(x, W, a)

    fn_row = fn.reshape(1, n)

    out = pl.pallas_call(
        _attn_kernel,
        grid=(n_row,),
        in_specs=[
            pl.BlockSpec((bm, n), lambda i: (i, 0)),
            pl.BlockSpec((bm, 1), lambda i: (i, 0)),
            pl.BlockSpec((1, n), lambda i: (0, 0)),
            pl.BlockSpec((1, 1), lambda i: (0, 0)),
            pl.BlockSpec((n, f_out), lambda i: (0, 0)),
        ],
        out_specs=pl.BlockSpec((bm, f_out), lambda i: (i, 0)),
        out_shape=jax.ShapeDtypeStruct((n, f_out), jnp.float32),
        compiler_params=pltpu.CompilerParams(
            dimension_semantics=("arbitrary",)),
    )(adj, fc, fn_row, mx, h)
    return out


def _pick_block(n, cap):
    best = 8
    for b in range(8, cap + 1, 8):
        if n % b == 0:
            best = b
    return best


def kernel(input, adj, W, a):
    n = input.shape[0]
    bm = _pick_block(n, 200)
    return _gat(input, adj, W, a, bm)


# speedup vs baseline: 2.4766x; 1.0664x over previous
import jax
import jax.numpy as jnp
from jax.experimental import pallas as pl
from jax.experimental.pallas import tpu as pltpu

_ALPHA = 0.2
_NEG_BIG = -1e30


def _proj_kernel(x_ref, w_ref, a_ref, h_ref, fc_ref, fn_ref, mx_ref):
    i = pl.program_id(0)
    h = jnp.dot(x_ref[...], w_ref[...], preferred_element_type=jnp.float32)
    h_ref[...] = h
    f = a_ref.shape[0] // 2
    fn = jnp.dot(h, a_ref[:f, :], preferred_element_type=jnp.float32)
    fn_ref[...] = fn
    fc_ref[...] = jnp.dot(h, a_ref[f:, :], preferred_element_type=jnp.float32)

    @pl.when(i == 0)
    def _init():
        mx_ref[...] = jnp.full_like(mx_ref, _NEG_BIG)

    mx_ref[...] = jnp.maximum(mx_ref[...], jnp.max(fn))


def _vec_kernel(fc_ref, fnr_ref, mx_ref, u_ref, up_ref, v_ref, vp_ref):
    mx = mx_ref[...]
    b = fc_ref[...] + mx
    t = jnp.maximum(b, _ALPHA * b)
    u_ref[...] = jnp.exp(b - t)
    up_ref[...] = jnp.exp(_ALPHA * b - t)
    d = fnr_ref[...] - mx
    v_ref[...] = jnp.exp(d)
    vp_ref[...] = jnp.exp(_ALPHA * d)


def _attn_kernel(adj_ref, u_ref, up_ref, v_ref, vp_ref, h_ref, out_ref):
    u = u_ref[...]
    up = up_ref[...]
    p = (jnp.maximum(u * v_ref[...], up * vp_ref[...])
         * adj_ref[...].astype(jnp.float32))
    s = jnp.sum(p, axis=1, keepdims=True)
    z = jnp.dot(p, h_ref[...], preferred_element_type=jnp.float32)
    z = z / jnp.where(s > 0, s, 1.0)
    out_ref[...] = jnp.where(z > 0, z, jnp.exp(jnp.minimum(z, 0.0)) - 1.0)


def _gat(x, adj, W, a, bm):
    n, f_in = x.shape
    f_out = W.shape[1]
    n_row = n // bm

    h, fc, fn, mx = pl.pallas_call(
        _proj_kernel,
        grid=(n_row,),
        in_specs=[
            pl.BlockSpec((bm, f_in), lambda i: (i, 0)),
            pl.BlockSpec((f_in, f_out), lambda i: (0, 0)),
            pl.BlockSpec((2 * f_out, 1), lambda i: (0, 0)),
        ],
        out_specs=[
            pl.BlockSpec((bm, f_out), lambda i: (i, 0)),
            pl.BlockSpec((bm, 1), lambda i: (i, 0)),
            pl.BlockSpec((bm, 1), lambda i: (i, 0)),
            pl.BlockSpec((1, 1), lambda i: (0, 0)),
        ],
        out_shape=[
            jax.ShapeDtypeStruct((n, f_out), jnp.float32),
            jax.ShapeDtypeStruct((n, 1), jnp.float32),
            jax.ShapeDtypeStruct((n, 1), jnp.float32),
            jax.ShapeDtypeStruct((1, 1), jnp.float32),
        ],
        compiler_params=pltpu.CompilerParams(
            dimension_semantics=("arbitrary",)),
    )(x, W, a)

    fn_row = fn.reshape(1, n)

    u, up, v, vp = pl.pallas_call(
        _vec_kernel,
        out_shape=[
            jax.ShapeDtypeStruct((n, 1), jnp.float32),
            jax.ShapeDtypeStruct((n, 1), jnp.float32),
            jax.ShapeDtypeStruct((1, n), jnp.float32),
            jax.ShapeDtypeStruct((1, n), jnp.float32),
        ],
    )(fc, fn_row, mx)

    out = pl.pallas_call(
        _attn_kernel,
        grid=(n_row,),
        in_specs=[
            pl.BlockSpec((bm, n), lambda i: (i, 0)),
            pl.BlockSpec((bm, 1), lambda i: (i, 0)),
            pl.BlockSpec((bm, 1), lambda i: (i, 0)),
            pl.BlockSpec((1, n), lambda i: (0, 0)),
            pl.BlockSpec((1, n), lambda i: (0, 0)),
            pl.BlockSpec((n, f_out), lambda i: (0, 0)),
        ],
        out_specs=pl.BlockSpec((bm, f_out), lambda i: (i, 0)),
        out_shape=jax.ShapeDtypeStruct((n, f_out), jnp.float32),
        compiler_params=pltpu.CompilerParams(
            dimension_semantics=("arbitrary",)),
    )(adj, u, up, v, vp, h)
    return out


def _pick_block(n, cap):
    best = 8
    for b in range(8, cap + 1, 8):
        if n % b == 0:
            best = b
    return best


def kernel(input, adj, W, a):
    n = input.shape[0]
    bm = _pick_block(n, 200)
    return _gat(input, adj, W, a, bm)


# speedup vs baseline: 2.8569x; 1.1535x over previous
import jax
import jax.numpy as jnp
from jax.experimental import pallas as pl
from jax.experimental.pallas import tpu as pltpu

_ALPHA = 0.2
_NEG_BIG = -1e30


def _proj_kernel(x_ref, w_ref, a_ref, haug_ref, fc_ref, fn_ref, mx_ref):
    i = pl.program_id(0)
    h = jnp.dot(x_ref[...], w_ref[...], preferred_element_type=jnp.float32)
    f = a_ref.shape[0] // 2
    fn = jnp.dot(h, a_ref[:f, :], preferred_element_type=jnp.float32)
    fn_ref[...] = fn
    fc_ref[...] = jnp.dot(h, a_ref[f:, :], preferred_element_type=jnp.float32)

    haug_ref[:, :f] = h.astype(jnp.bfloat16)
    lane = jax.lax.broadcasted_iota(jnp.int32, (h.shape[0], f), 1)
    haug_ref[:, f:] = jnp.where(lane == 0, 1.0, 0.0).astype(jnp.bfloat16)

    @pl.when(i == 0)
    def _init():
        mx_ref[...] = jnp.full_like(mx_ref, _NEG_BIG)

    mx_ref[...] = jnp.maximum(mx_ref[...], jnp.max(fn))


def _vec_kernel(fc_ref, fnr_ref, mx_ref, u_ref, up_ref, v_ref, vp_ref):
    mx = mx_ref[...]
    b = fc_ref[...] + mx
    t = jnp.maximum(b, _ALPHA * b)
    u_ref[...] = jnp.exp(b - t)
    up_ref[...] = jnp.exp(_ALPHA * b - t)
    d = fnr_ref[...] - mx
    v_ref[...] = jnp.exp(d)
    vp_ref[...] = jnp.exp(_ALPHA * d)


def _attn_kernel(adj_ref, u_ref, up_ref, v_ref, vp_ref, haug_ref, out_ref):
    p = (jnp.maximum(u_ref[...] * v_ref[...], up_ref[...] * vp_ref[...])
         * adj_ref[...].astype(jnp.float32)).astype(jnp.bfloat16)
    zaug = jnp.dot(p, haug_ref[...], preferred_element_type=jnp.float32)
    f = out_ref.shape[1]
    z = zaug[:, :f]
    s = zaug[:, f:f + 1]
    z = z / jnp.where(s > 0, s, 1.0)
    out_ref[...] = jnp.where(z > 0, z, jnp.exp(jnp.minimum(z, 0.0)) - 1.0)


def _gat(x, adj, W, a, bm):
    n, f_in = x.shape
    f_out = W.shape[1]
    n_row = n // bm

    haug, fc, fn, mx = pl.pallas_call(
        _proj_kernel,
        grid=(n_row,),
        in_specs=[
            pl.BlockSpec((bm, f_in), lambda i: (i, 0)),
            pl.BlockSpec((f_in, f_out), lambda i: (0, 0)),
            pl.BlockSpec((2 * f_out, 1), lambda i: (0, 0)),
        ],
        out_specs=[
            pl.BlockSpec((bm, 2 * f_out), lambda i: (i, 0)),
            pl.BlockSpec((bm, 1), lambda i: (i, 0)),
            pl.BlockSpec((bm, 1), lambda i: (i, 0)),
            pl.BlockSpec((1, 1), lambda i: (0, 0)),
        ],
        out_shape=[
            jax.ShapeDtypeStruct((n, 2 * f_out), jnp.bfloat16),
            jax.ShapeDtypeStruct((n, 1), jnp.float32),
            jax.ShapeDtypeStruct((n, 1), jnp.float32),
            jax.ShapeDtypeStruct((1, 1), jnp.float32),
        ],
        compiler_params=pltpu.CompilerParams(
            dimension_semantics=("arbitrary",)),
    )(x, W, a)

    fn_row = fn.reshape(1, n)

    u, up, v, vp = pl.pallas_call(
        _vec_kernel,
        out_shape=[
            jax.ShapeDtypeStruct((n, 1), jnp.float32),
            jax.ShapeDtypeStruct((n, 1), jnp.float32),
            jax.ShapeDtypeStruct((1, n), jnp.float32),
            jax.ShapeDtypeStruct((1, n), jnp.float32),
        ],
    )(fc, fn_row, mx)

    out = pl.pallas_call(
        _attn_kernel,
        grid=(n_row,),
        in_specs=[
            pl.BlockSpec((bm, n), lambda i: (i, 0)),
            pl.BlockSpec((bm, 1), lambda i: (i, 0)),
            pl.BlockSpec((bm, 1), lambda i: (i, 0)),
            pl.BlockSpec((1, n), lambda i: (0, 0)),
            pl.BlockSpec((1, n), lambda i: (0, 0)),
            pl.BlockSpec((n, 2 * f_out), lambda i: (0, 0)),
        ],
        out_specs=pl.BlockSpec((bm, f_out), lambda i: (i, 0)),
        out_shape=jax.ShapeDtypeStruct((n, f_out), jnp.float32),
        compiler_params=pltpu.CompilerParams(
            dimension_semantics=("arbitrary",)),
    )(adj, u, up, v, vp, haug)
    return out


def _pick_block(n, cap):
    best = 8
    for b in range(8, cap + 1, 8):
        if n % b == 0:
            best = b
    return best


def kernel(input, adj, W, a):
    n = input.shape[0]
    bm = _pick_block(n, 200)
    return _gat(input, adj, W, a, bm)


# speedup vs baseline: 3.2668x; 1.1435x over previous
import jax
import jax.numpy as jnp
from jax.experimental import pallas as pl
from jax.experimental.pallas import tpu as pltpu

_ALPHA = 0.2
_NEG_BIG = -1e30


def _proj_kernel(x_ref, w_ref, a_ref, haug_ref, fc_ref, fn_ref, mx_ref):
    i = pl.program_id(0)
    h = jnp.dot(x_ref[...], w_ref[...], preferred_element_type=jnp.float32)
    f = a_ref.shape[0] // 2
    fn = jnp.dot(h, a_ref[:f, :], preferred_element_type=jnp.float32)
    fn_ref[...] = fn
    fc_ref[...] = jnp.dot(h, a_ref[f:, :], preferred_element_type=jnp.float32)

    haug_ref[:, :f] = h.astype(jnp.bfloat16)
    lane = jax.lax.broadcasted_iota(jnp.int32, (h.shape[0], f), 1)
    haug_ref[:, f:] = jnp.where(lane == 0, 1.0, 0.0).astype(jnp.bfloat16)

    @pl.when(i == 0)
    def _init():
        mx_ref[...] = jnp.full_like(mx_ref, _NEG_BIG)

    mx_ref[...] = jnp.maximum(mx_ref[...], jnp.max(fn))


def _vec_kernel(fc_ref, fnr_ref, mx_ref, u_ref, up_ref, v_ref, vp_ref):
    mx = mx_ref[...]
    b = fc_ref[...] + mx
    t = jnp.maximum(b, _ALPHA * b)
    u_ref[...] = jnp.exp(b - t)
    up_ref[...] = jnp.exp(_ALPHA * b - t)
    d = fnr_ref[...] - mx
    v_ref[...] = jnp.exp(d)
    vp_ref[...] = jnp.exp(_ALPHA * d)


def _attn_kernel(adj_ref, u_ref, up_ref, v_ref, vp_ref, haug_ref, out_ref):
    p = (jnp.maximum(u_ref[...] * v_ref[...], up_ref[...] * vp_ref[...])
         * adj_ref[...].astype(jnp.float32)).astype(jnp.bfloat16)
    zaug = jnp.dot(p, haug_ref[...], preferred_element_type=jnp.float32)
    f = out_ref.shape[1]
    z = zaug[:, :f]
    s = zaug[:, f:f + 1]
    z = z / jnp.where(s > 0, s, 1.0)
    out_ref[...] = jnp.where(z > 0, z, jnp.exp(jnp.minimum(z, 0.0)) - 1.0)


def _gat(x, adj, W, a, bm):
    n, f_in = x.shape
    f_out = W.shape[1]
    n_row = n // bm

    haug, fc, fn, mx = pl.pallas_call(
        _proj_kernel,
        grid=(n_row,),
        in_specs=[
            pl.BlockSpec((bm, f_in), lambda i: (i, 0)),
            pl.BlockSpec((f_in, f_out), lambda i: (0, 0)),
            pl.BlockSpec((2 * f_out, 1), lambda i: (0, 0)),
        ],
        out_specs=[
            pl.BlockSpec((bm, 2 * f_out), lambda i: (i, 0)),
            pl.BlockSpec((bm, 1), lambda i: (i, 0)),
            pl.BlockSpec((bm, 1), lambda i: (i, 0)),
            pl.BlockSpec((1, 1), lambda i: (0, 0)),
        ],
        out_shape=[
            jax.ShapeDtypeStruct((n, 2 * f_out), jnp.bfloat16),
            jax.ShapeDtypeStruct((n, 1), jnp.float32),
            jax.ShapeDtypeStruct((n, 1), jnp.float32),
            jax.ShapeDtypeStruct((1, 1), jnp.float32),
        ],
        compiler_params=pltpu.CompilerParams(
            dimension_semantics=("arbitrary",)),
    )(x, W, a)

    fn_row = fn.reshape(1, n)

    u, up, v, vp = pl.pallas_call(
        _vec_kernel,
        out_shape=[
            jax.ShapeDtypeStruct((n, 1), jnp.float32),
            jax.ShapeDtypeStruct((n, 1), jnp.float32),
            jax.ShapeDtypeStruct((1, n), jnp.float32),
            jax.ShapeDtypeStruct((1, n), jnp.float32),
        ],
    )(fc, fn_row, mx)

    out = pl.pallas_call(
        _attn_kernel,
        grid=(n_row,),
        in_specs=[
            pl.BlockSpec((bm, n), lambda i: (i, 0)),
            pl.BlockSpec((bm, 1), lambda i: (i, 0)),
            pl.BlockSpec((bm, 1), lambda i: (i, 0)),
            pl.BlockSpec((1, n), lambda i: (0, 0)),
            pl.BlockSpec((1, n), lambda i: (0, 0)),
            pl.BlockSpec((n, 2 * f_out), lambda i: (0, 0)),
        ],
        out_specs=pl.BlockSpec((bm, f_out), lambda i: (i, 0)),
        out_shape=jax.ShapeDtypeStruct((n, f_out), jnp.float32),
        compiler_params=pltpu.CompilerParams(
            dimension_semantics=("arbitrary",)),
    )(adj, u, up, v, vp, haug)
    return out


def _pick_block(n, cap):
    best = 8
    for b in range(8, cap + 1, 8):
        if n % b == 0:
            best = b
    return best


def kernel(input, adj, W, a):
    n = input.shape[0]
    bm = _pick_block(n, 400)
    return _gat(input, adj, W, a, bm)


# speedup vs baseline: 3.2689x; 1.0006x over previous
import jax
import jax.numpy as jnp
from jax.experimental import pallas as pl
from jax.experimental.pallas import tpu as pltpu

_ALPHA = 0.2
_NEG_BIG = -1e30


def _proj_kernel(x_ref, w_ref, a_ref, haug_ref, fc_ref, fn_ref, mx_ref):
    i = pl.program_id(0)
    h = jnp.dot(x_ref[...], w_ref[...], preferred_element_type=jnp.float32)
    f = a_ref.shape[0] // 2
    fn = jnp.dot(h, a_ref[:f, :], preferred_element_type=jnp.float32)
    fn_ref[...] = fn
    fc_ref[...] = jnp.dot(h, a_ref[f:, :], preferred_element_type=jnp.float32)

    haug_ref[:, :f] = h.astype(jnp.bfloat16)
    lane = jax.lax.broadcasted_iota(jnp.int32, (h.shape[0], f), 1)
    haug_ref[:, f:] = jnp.where(lane == 0, 1.0, 0.0).astype(jnp.bfloat16)

    @pl.when(i == 0)
    def _init():
        mx_ref[...] = jnp.full_like(mx_ref, _NEG_BIG)

    mx_ref[...] = jnp.maximum(mx_ref[...], jnp.max(fn))


def _vec_kernel(fc_ref, fnr_ref, mx_ref, u_ref, up_ref, v_ref, vp_ref):
    mx = mx_ref[...]
    b = fc_ref[...] + mx
    t = jnp.maximum(b, _ALPHA * b)
    u_ref[...] = jnp.exp(b - t)
    up_ref[...] = jnp.exp(_ALPHA * b - t)
    d = fnr_ref[...] - mx
    v_ref[...] = jnp.exp(d)
    vp_ref[...] = jnp.exp(_ALPHA * d)


def _attn_kernel(adj_ref, u_ref, up_ref, v_ref, vp_ref, haug_ref, out_ref):
    p = (jnp.maximum(u_ref[...] * v_ref[...], up_ref[...] * vp_ref[...])
         * adj_ref[...].astype(jnp.float32)).astype(jnp.bfloat16)
    zaug = jnp.dot(p, haug_ref[...], preferred_element_type=jnp.float32)
    f = out_ref.shape[1]
    z = zaug[:, :f]
    s = zaug[:, f:f + 1]
    z = z / jnp.where(s > 0, s, 1.0)
    out_ref[...] = jnp.where(z > 0, z, jnp.exp(jnp.minimum(z, 0.0)) - 1.0)


def _gat(x, adj, W, a, bm):
    n, f_in = x.shape
    f_out = W.shape[1]
    n_row = n // bm

    haug, fc, fn, mx = pl.pallas_call(
        _proj_kernel,
        grid=(n_row,),
        in_specs=[
            pl.BlockSpec((bm, f_in), lambda i: (i, 0)),
            pl.BlockSpec((f_in, f_out), lambda i: (0, 0)),
            pl.BlockSpec((2 * f_out, 1), lambda i: (0, 0)),
        ],
        out_specs=[
            pl.BlockSpec((bm, 2 * f_out), lambda i: (i, 0)),
            pl.BlockSpec((bm, 1), lambda i: (i, 0)),
            pl.BlockSpec((bm, 1), lambda i: (i, 0)),
            pl.BlockSpec((1, 1), lambda i: (0, 0)),
        ],
        out_shape=[
            jax.ShapeDtypeStruct((n, 2 * f_out), jnp.bfloat16),
            jax.ShapeDtypeStruct((n, 1), jnp.float32),
            jax.ShapeDtypeStruct((n, 1), jnp.float32),
            jax.ShapeDtypeStruct((1, 1), jnp.float32),
        ],
        compiler_params=pltpu.CompilerParams(
            dimension_semantics=("arbitrary",)),
    )(x, W, a)

    fn_row = fn.reshape(1, n)

    u, up, v, vp = pl.pallas_call(
        _vec_kernel,
        out_shape=[
            jax.ShapeDtypeStruct((n, 1), jnp.float32),
            jax.ShapeDtypeStruct((n, 1), jnp.float32),
            jax.ShapeDtypeStruct((1, n), jnp.float32),
            jax.ShapeDtypeStruct((1, n), jnp.float32),
        ],
    )(fc, fn_row, mx)

    out = pl.pallas_call(
        _attn_kernel,
        grid=(n_row,),
        in_specs=[
            pl.BlockSpec((bm, n), lambda i: (i, 0)),
            pl.BlockSpec((bm, 1), lambda i: (i, 0)),
            pl.BlockSpec((bm, 1), lambda i: (i, 0)),
            pl.BlockSpec((1, n), lambda i: (0, 0)),
            pl.BlockSpec((1, n), lambda i: (0, 0)),
            pl.BlockSpec((n, 2 * f_out), lambda i: (0, 0)),
        ],
        out_specs=pl.BlockSpec((bm, f_out), lambda i: (i, 0)),
        out_shape=jax.ShapeDtypeStruct((n, f_out), jnp.float32),
        compiler_params=pltpu.CompilerParams(
            dimension_semantics=("arbitrary",),
            vmem_limit_bytes=100 * 1024 * 1024),
    )(adj, u, up, v, vp, haug)
    return out


def _pick_block(n, cap):
    best = 8
    for b in range(8, cap + 1, 8):
        if n % b == 0:
            best = b
    return best


def kernel(input, adj, W, a):
    n = input.shape[0]
    bm = _pick_block(n, 400)
    return _gat(input, adj, W, a, bm)
